# baseline (speedup 1.0000x reference)
import functools

import jax
import jax.numpy as jnp
from jax import lax
from jax.experimental import pallas as pl
from jax.experimental.pallas import tpu as pltpu
from jax.experimental.pallas import tpu_sc as plsc

N = 10000
E = 160000
D = 256
H = 256
G = 64
L = 3
HC = H * L
HHALF = 128

NC = 2
NS = 16
NPAD = 10240
ROWS_PER_TILE = NPAD // NS
CHUNK = 128
NROWS2D = 1280
ROWS_MAIN = NROWS2D // NS
EPAD = NROWS2D * CHUNK - E

BLK = 1000


STRIPE = 8
NSTRIPES = ROWS_MAIN // STRIPE


def _sc_agg_body(z2, src2, dst2, zeros_h, out,
                 acc, ss0, ss1, ds0, ds1, rowsb,
                 semls0, semls1, semld0, semld1, semg):
    sbufs = (ss0, ss1)
    dbufs = (ds0, ds1)
    semls = (semls0, semls1)
    semld = (semld0, semld1)
    c = lax.axis_index("c")
    s = lax.axis_index("s")
    row0 = s * ROWS_PER_TILE
    pltpu.sync_copy(zeros_h.at[pl.ds(row0, ROWS_PER_TILE)],
                    acc.at[pl.ds(row0, ROWS_PER_TILE)])
    plsc.subcore_barrier()

    r0base = s * ROWS_MAIN

    def fire_load(k, p):
        base = r0base + k * STRIPE
        pltpu.async_copy(src2.at[pl.ds(base, STRIPE)], sbufs[p], semls[p])
        pltpu.async_copy(dst2.at[pl.ds(base, STRIPE)], dbufs[p], semld[p])

    def wait_load(k, p):
        base = r0base + k * STRIPE
        pltpu.make_async_copy(src2.at[pl.ds(base, STRIPE)], sbufs[p],
                              semls[p]).wait()
        pltpu.make_async_copy(dst2.at[pl.ds(base, STRIPE)], dbufs[p],
                              semld[p]).wait()

    def transform(p):
        ss = sbufs[p]
        for r in range(STRIPE):
            for i in range(CHUNK // 16):
                sl = pl.ds(i * 16, 16)
                ss[r, sl] = ss[r, sl] * 2 + c

    def process_stripe(k, p):
        ss, ds = sbufs[p], dbufs[p]
        pn = 1 - p
        for r in range(STRIPE):
            pltpu.async_copy(z2.at[ss.at[r]], rowsb, semg).wait()
            pltpu.sync_copy(rowsb, acc.at[ds.at[r]], add=True)

        @pl.when(k + 1 < NSTRIPES)
        def _():
            wait_load(k + 1, pn)
            transform(pn)

        @pl.when(k + 2 < NSTRIPES)
        def _():
            fire_load(k + 2, p)

    pltpu.sync_copy(src2.at[pl.ds(r0base, STRIPE)], ss0)
    pltpu.sync_copy(dst2.at[pl.ds(r0base, STRIPE)], ds0)
    fire_load(1, 1)
    transform(0)

    def outer(o, carry):
        process_stripe(2 * o, 0)
        process_stripe(2 * o + 1, 1)
        return carry

    lax.fori_loop(0, NSTRIPES // 2, outer, 0)

    plsc.subcore_barrier()

    pltpu.sync_copy(acc.at[pl.ds(row0, ROWS_PER_TILE)],
                    out.at[c, pl.ds(row0, ROWS_PER_TILE)])


@functools.lru_cache(maxsize=None)
def _sc_agg_kernel():
    return pl.kernel(
        _sc_agg_body,
        out_type=jax.ShapeDtypeStruct((NC, NPAD, HHALF), jnp.float32),
        mesh=plsc.VectorSubcoreMesh(core_axis_name="c", subcore_axis_name="s",
                                    num_cores=NC, num_subcores=NS),
        scratch_types=[
            pltpu.VMEM_SHARED((NPAD, HHALF), jnp.float32),
            pltpu.VMEM((STRIPE, CHUNK), jnp.int32),
            pltpu.VMEM((STRIPE, CHUNK), jnp.int32),
            pltpu.VMEM((STRIPE, CHUNK), jnp.int32),
            pltpu.VMEM((STRIPE, CHUNK), jnp.int32),
            pltpu.VMEM((CHUNK, HHALF), jnp.float32),
            pltpu.SemaphoreType.DMA,
            pltpu.SemaphoreType.DMA,
            pltpu.SemaphoreType.DMA,
            pltpu.SemaphoreType.DMA,
            pltpu.SemaphoreType.DMA,
        ],
    )


def _mlp3_body(z_ref, lo_ref, hi_ref, w1_ref, b1_ref,
               w20_ref, w21_ref, w22_ref, b2_ref, h_ref, st_ref):
    i = pl.program_id(0)
    u = z_ref[...] + jnp.concatenate([lo_ref[...], hi_ref[...]], axis=1)
    t = jnp.maximum(
        jnp.dot(u, w1_ref[...], preferred_element_type=jnp.float32)
        + b1_ref[...], 0.0)
    hs = []
    for li, w2_ref in enumerate((w20_ref, w21_ref, w22_ref)):
        ti = t[:, li * H:(li + 1) * H]
        hs.append(jnp.maximum(
            jnp.dot(ti, w2_ref[...], preferred_element_type=jnp.float32)
            + b2_ref[:, li * H:(li + 1) * H], 0.0))
    h = jnp.concatenate(hs, axis=1)
    h_ref[...] = h

    @pl.when(i == 0)
    def _():
        st_ref[...] = jnp.zeros_like(st_ref)

    st_ref[0:1, :] += jnp.sum(h, axis=0, keepdims=True)
    st_ref[1:2, :] += jnp.sum(h * h, axis=0, keepdims=True)


def _mlp3(z, agg_lo, agg_hi, w1c, b1c, w20, w21, w22, b2c):
    return pl.pallas_call(
        _mlp3_body,
        grid=(N // BLK,),
        in_specs=[
            pl.BlockSpec((BLK, D), lambda i: (i, 0)),
            pl.BlockSpec((BLK, HHALF), lambda i: (i, 0)),
            pl.BlockSpec((BLK, HHALF), lambda i: (i, 0)),
            pl.BlockSpec((D, HC), lambda i: (0, 0)),
            pl.BlockSpec((1, HC), lambda i: (0, 0)),
            pl.BlockSpec((H, H), lambda i: (0, 0)),
            pl.BlockSpec((H, H), lambda i: (0, 0)),
            pl.BlockSpec((H, H), lambda i: (0, 0)),
            pl.BlockSpec((1, HC), lambda i: (0, 0)),
        ],
        out_specs=[
            pl.BlockSpec((BLK, HC), lambda i: (i, 0)),
            pl.BlockSpec((8, HC), lambda i: (0, 0)),
        ],
        out_shape=[
            jax.ShapeDtypeStruct((N, HC), jnp.float32),
            jax.ShapeDtypeStruct((8, HC), jnp.float32),
        ],
    )(z, agg_lo, agg_hi, w1c, b1c, w20, w21, w22, b2c)


def _norm_pool_body(h_ref, st_ref, g_ref, b_ref, oh_ref, hbn_ref, pool_ref):
    i = pl.program_id(0)
    st = st_ref[...]
    mean = st[0:1, :] * (1.0 / N)
    var = st[1:2, :] * (1.0 / N) - mean * mean
    rstd = lax.rsqrt(var + 1e-5)
    hbn = (h_ref[...] - mean) * (rstd * g_ref[...]) + b_ref[...]
    hbn_ref[...] = hbn

    @pl.when(i == 0)
    def _():
        pool_ref[...] = jnp.zeros_like(pool_ref)

    pool_ref[...] += lax.dot_general(
        oh_ref[...], hbn, (((0,), (0,)), ((), ())),
        preferred_element_type=jnp.float32)


def _norm_pool(h, st, gamma, beta, onehot):
    return pl.pallas_call(
        _norm_pool_body,
        grid=(N // BLK,),
        in_specs=[
            pl.BlockSpec((BLK, HC), lambda i: (i, 0)),
            pl.BlockSpec((8, HC), lambda i: (0, 0)),
            pl.BlockSpec((1, HC), lambda i: (0, 0)),
            pl.BlockSpec((1, HC), lambda i: (0, 0)),
            pl.BlockSpec((BLK, G), lambda i: (i, 0)),
        ],
        out_specs=[
            pl.BlockSpec((BLK, HC), lambda i: (i, 0)),
            pl.BlockSpec((G, HC), lambda i: (0, 0)),
        ],
        out_shape=[
            jax.ShapeDtypeStruct((N, HC), jnp.float32),
            jax.ShapeDtypeStruct((G, HC), jnp.float32),
        ],
    )(h, st, gamma, beta, onehot)


def kernel(x, edge_index, batch, w1_0, b1_0, w2_0, b2_0, gamma_0, beta_0,
           w1_1, b1_1, w2_1, b2_1, gamma_1, beta_1,
           w1_2, b1_2, w2_2, b2_2, gamma_2, beta_2):
    src = edge_index[0]
    dst = edge_index[1]
    zeros_pad = jnp.zeros((NPAD, HHALF), jnp.float32)
    onehot = (batch[:, None] == jnp.arange(G, dtype=batch.dtype)[None, :]
              ).astype(jnp.float32)

    src_pad = jnp.concatenate(
        [src, jnp.zeros((EPAD,), jnp.int32)]).reshape(NROWS2D, CHUNK)
    dst_pad = jnp.concatenate(
        [dst, jnp.full((EPAD,), NPAD - 1, jnp.int32)]).reshape(NROWS2D, CHUNK)
    agg2 = _sc_agg_kernel()(x.reshape(2 * N, HHALF), src_pad, dst_pad,
                            zeros_pad)

    w1c = jnp.concatenate([w1_0, w1_1, w1_2], axis=1)
    b1c = jnp.concatenate([b1_0, b1_1, b1_2]).reshape(1, HC)
    b2c = jnp.concatenate([b2_0, b2_1, b2_2]).reshape(1, HC)
    gmc = jnp.concatenate([gamma_0, gamma_1, gamma_2]).reshape(1, HC)
    btc = jnp.concatenate([beta_0, beta_1, beta_2]).reshape(1, HC)

    h_cat, st = _mlp3(x, agg2[0, :N], agg2[1, :N],
                      w1c, b1c, w2_0, w2_1, w2_2, b2c)
    z_cat, g_cat = _norm_pool(h_cat, st, gmc, btc, onehot)
    return z_cat, g_cat


# speedup vs baseline: 1.3150x; 1.3150x over previous
import functools

import jax
import jax.numpy as jnp
from jax import lax
from jax.experimental import pallas as pl
from jax.experimental.pallas import tpu as pltpu
from jax.experimental.pallas import tpu_sc as plsc

N = 10000
E = 160000
D = 256
H = 256
G = 64
L = 3
HC = H * L
HHALF = 128

NC = 2
NS = 16
NPAD = 10240
ROWS_PER_TILE = NPAD // NS
EDGES_PER_SUB = E // NS
CHUNK = 128
NFULL = EDGES_PER_SUB // CHUNK
TAIL = EDGES_PER_SUB - NFULL * CHUNK

BLK = 1000


def _sc_agg_body(z2, src_h, dst_h, zeros_h, out,
                 acc, src_v, gidx_v, dst_v, rows_v,
                 src_t, gidx_t, dst_t, rows_t, sem):
    c = lax.axis_index("c")
    s = lax.axis_index("s")
    row0 = s * ROWS_PER_TILE
    pltpu.sync_copy(zeros_h.at[pl.ds(row0, ROWS_PER_TILE)],
                    acc.at[pl.ds(row0, ROWS_PER_TILE)])
    plsc.subcore_barrier()

    base0 = s * EDGES_PER_SUB

    def chunk(j, carry):
        base = base0 + j * CHUNK
        pltpu.sync_copy(src_h.at[pl.ds(base, CHUNK)], src_v)
        for i in range(CHUNK // 16):
            sl = pl.ds(i * 16, 16)
            gidx_v[sl] = src_v[sl] * 2 + c
        pltpu.async_copy(z2.at[gidx_v], rows_v, sem).wait()
        pltpu.sync_copy(dst_h.at[pl.ds(base, CHUNK)], dst_v)
        pltpu.sync_copy(rows_v, acc.at[dst_v], add=True)
        return carry

    lax.fori_loop(0, NFULL, chunk, 0)

    base = base0 + NFULL * CHUNK
    pltpu.sync_copy(src_h.at[pl.ds(base, TAIL)], src_t)
    gidx_t[...] = src_t[...] * 2 + c
    pltpu.async_copy(z2.at[gidx_t], rows_t, sem).wait()
    pltpu.sync_copy(dst_h.at[pl.ds(base, TAIL)], dst_t)
    pltpu.sync_copy(rows_t, acc.at[dst_t], add=True)

    plsc.subcore_barrier()

    pltpu.sync_copy(acc.at[pl.ds(row0, ROWS_PER_TILE)],
                    out.at[c, pl.ds(row0, ROWS_PER_TILE)])


@functools.lru_cache(maxsize=None)
def _sc_agg_kernel():
    return pl.kernel(
        _sc_agg_body,
        out_type=jax.ShapeDtypeStruct((NC, NPAD, HHALF), jnp.float32),
        mesh=plsc.VectorSubcoreMesh(core_axis_name="c", subcore_axis_name="s",
                                    num_cores=NC, num_subcores=NS),
        scratch_types=[
            pltpu.VMEM_SHARED((NPAD, HHALF), jnp.float32),
            pltpu.VMEM((CHUNK,), jnp.int32),
            pltpu.VMEM((CHUNK,), jnp.int32),
            pltpu.VMEM((CHUNK,), jnp.int32),
            pltpu.VMEM((CHUNK, HHALF), jnp.float32),
            pltpu.VMEM((TAIL,), jnp.int32),
            pltpu.VMEM((TAIL,), jnp.int32),
            pltpu.VMEM((TAIL,), jnp.int32),
            pltpu.VMEM((TAIL, HHALF), jnp.float32),
            pltpu.SemaphoreType.DMA,
        ],
    )


def _mlp3_body(z_ref, lo_ref, hi_ref, w1_ref, b1_ref,
               w20_ref, w21_ref, w22_ref, b2_ref, h_ref, st_ref):
    i = pl.program_id(0)
    u = z_ref[...] + jnp.concatenate([lo_ref[...], hi_ref[...]], axis=1)
    t = jnp.maximum(
        jnp.dot(u, w1_ref[...], preferred_element_type=jnp.float32)
        + b1_ref[...], 0.0)
    hs = []
    for li, w2_ref in enumerate((w20_ref, w21_ref, w22_ref)):
        ti = t[:, li * H:(li + 1) * H]
        hs.append(jnp.maximum(
            jnp.dot(ti, w2_ref[...], preferred_element_type=jnp.float32)
            + b2_ref[:, li * H:(li + 1) * H], 0.0))
    h = jnp.concatenate(hs, axis=1)
    h_ref[...] = h

    @pl.when(i == 0)
    def _():
        st_ref[...] = jnp.zeros_like(st_ref)

    st_ref[0:1, :] += jnp.sum(h, axis=0, keepdims=True)
    st_ref[1:2, :] += jnp.sum(h * h, axis=0, keepdims=True)


def _mlp3(z, agg_lo, agg_hi, w1c, b1c, w20, w21, w22, b2c):
    return pl.pallas_call(
        _mlp3_body,
        grid=(N // BLK,),
        in_specs=[
            pl.BlockSpec((BLK, D), lambda i: (i, 0)),
            pl.BlockSpec((BLK, HHALF), lambda i: (i, 0)),
            pl.BlockSpec((BLK, HHALF), lambda i: (i, 0)),
            pl.BlockSpec((D, HC), lambda i: (0, 0)),
            pl.BlockSpec((1, HC), lambda i: (0, 0)),
            pl.BlockSpec((H, H), lambda i: (0, 0)),
            pl.BlockSpec((H, H), lambda i: (0, 0)),
            pl.BlockSpec((H, H), lambda i: (0, 0)),
            pl.BlockSpec((1, HC), lambda i: (0, 0)),
        ],
        out_specs=[
            pl.BlockSpec((BLK, HC), lambda i: (i, 0)),
            pl.BlockSpec((8, HC), lambda i: (0, 0)),
        ],
        out_shape=[
            jax.ShapeDtypeStruct((N, HC), jnp.float32),
            jax.ShapeDtypeStruct((8, HC), jnp.float32),
        ],
    )(z, agg_lo, agg_hi, w1c, b1c, w20, w21, w22, b2c)


def _norm_pool_body(h_ref, st_ref, g_ref, b_ref, oh_ref, hbn_ref, pool_ref):
    i = pl.program_id(0)
    st = st_ref[...]
    mean = st[0:1, :] * (1.0 / N)
    var = st[1:2, :] * (1.0 / N) - mean * mean
    rstd = lax.rsqrt(var + 1e-5)
    hbn = (h_ref[...] - mean) * (rstd * g_ref[...]) + b_ref[...]
    hbn_ref[...] = hbn

    @pl.when(i == 0)
    def _():
        pool_ref[...] = jnp.zeros_like(pool_ref)

    pool_ref[...] += lax.dot_general(
        oh_ref[...], hbn, (((0,), (0,)), ((), ())),
        preferred_element_type=jnp.float32)


def _norm_pool(h, st, gamma, beta, onehot):
    return pl.pallas_call(
        _norm_pool_body,
        grid=(N // BLK,),
        in_specs=[
            pl.BlockSpec((BLK, HC), lambda i: (i, 0)),
            pl.BlockSpec((8, HC), lambda i: (0, 0)),
            pl.BlockSpec((1, HC), lambda i: (0, 0)),
            pl.BlockSpec((1, HC), lambda i: (0, 0)),
            pl.BlockSpec((BLK, G), lambda i: (i, 0)),
        ],
        out_specs=[
            pl.BlockSpec((BLK, HC), lambda i: (i, 0)),
            pl.BlockSpec((G, HC), lambda i: (0, 0)),
        ],
        out_shape=[
            jax.ShapeDtypeStruct((N, HC), jnp.float32),
            jax.ShapeDtypeStruct((G, HC), jnp.float32),
        ],
    )(h, st, gamma, beta, onehot)


def kernel(x, edge_index, batch, w1_0, b1_0, w2_0, b2_0, gamma_0, beta_0,
           w1_1, b1_1, w2_1, b2_1, gamma_1, beta_1,
           w1_2, b1_2, w2_2, b2_2, gamma_2, beta_2):
    src = edge_index[0]
    dst = edge_index[1]
    zeros_pad = jnp.zeros((NPAD, HHALF), jnp.float32)
    onehot = (batch[:, None] == jnp.arange(G, dtype=batch.dtype)[None, :]
              ).astype(jnp.float32)

    agg2 = _sc_agg_kernel()(x.reshape(2 * N, HHALF), src, dst, zeros_pad)

    w1c = jnp.concatenate([w1_0, w1_1, w1_2], axis=1)
    b1c = jnp.concatenate([b1_0, b1_1, b1_2]).reshape(1, HC)
    b2c = jnp.concatenate([b2_0, b2_1, b2_2]).reshape(1, HC)
    gmc = jnp.concatenate([gamma_0, gamma_1, gamma_2]).reshape(1, HC)
    btc = jnp.concatenate([beta_0, beta_1, beta_2]).reshape(1, HC)

    h_cat, st = _mlp3(x, agg2[0, :N], agg2[1, :N],
                      w1c, b1c, w2_0, w2_1, w2_2, b2c)
    z_cat, g_cat = _norm_pool(h_cat, st, gmc, btc, onehot)
    return z_cat, g_cat


# speedup vs baseline: 1.4657x; 1.1146x over previous
import functools

import jax
import jax.numpy as jnp
from jax import lax
from jax.experimental import pallas as pl
from jax.experimental.pallas import tpu as pltpu
from jax.experimental.pallas import tpu_sc as plsc

N = 10000
E = 160000
D = 256
H = 256
G = 64
L = 3
HC = H * L
HHALF = 128

NC = 2
NS = 16
NPAD = 10240
ROWS_PER_TILE = NPAD // NS
EDGES_PER_SUB = E // NS
CHUNK = 128
NFULL = EDGES_PER_SUB // CHUNK
TAIL = EDGES_PER_SUB - NFULL * CHUNK

BLK = 1000


def _sc_agg_body(z2, src_h, dst_h, zeros_h, out,
                 acc, src_v, gidx_v, dst_v, rows_v,
                 src_t, gidx_t, dst_t, rows_t, sem, semd):
    c = lax.axis_index("c")
    s = lax.axis_index("s")
    row0 = s * ROWS_PER_TILE
    pltpu.sync_copy(zeros_h.at[pl.ds(row0, ROWS_PER_TILE)],
                    acc.at[pl.ds(row0, ROWS_PER_TILE)])
    plsc.subcore_barrier()

    base0 = s * EDGES_PER_SUB

    def chunk(j, carry):
        base = base0 + j * CHUNK
        pltpu.sync_copy(src_h.at[pl.ds(base, CHUNK)], src_v)
        pltpu.async_copy(dst_h.at[pl.ds(base, CHUNK)], dst_v, semd)
        for i in range(CHUNK // 16):
            sl = pl.ds(i * 16, 16)
            gidx_v[sl] = src_v[sl] * 2 + c
        pltpu.async_copy(z2.at[gidx_v], rows_v, sem).wait()
        pltpu.make_async_copy(dst_h.at[pl.ds(base, CHUNK)], dst_v,
                              semd).wait()
        pltpu.sync_copy(rows_v, acc.at[dst_v], add=True)
        return carry

    lax.fori_loop(0, NFULL, chunk, 0)

    base = base0 + NFULL * CHUNK
    pltpu.sync_copy(src_h.at[pl.ds(base, TAIL)], src_t)
    gidx_t[...] = src_t[...] * 2 + c
    pltpu.async_copy(z2.at[gidx_t], rows_t, sem).wait()
    pltpu.sync_copy(dst_h.at[pl.ds(base, TAIL)], dst_t)
    pltpu.sync_copy(rows_t, acc.at[dst_t], add=True)

    plsc.subcore_barrier()

    pltpu.sync_copy(acc.at[pl.ds(row0, ROWS_PER_TILE)],
                    out.at[c, pl.ds(row0, ROWS_PER_TILE)])


@functools.lru_cache(maxsize=None)
def _sc_agg_kernel():
    return pl.kernel(
        _sc_agg_body,
        out_type=jax.ShapeDtypeStruct((NC, NPAD, HHALF), jnp.float32),
        mesh=plsc.VectorSubcoreMesh(core_axis_name="c", subcore_axis_name="s",
                                    num_cores=NC, num_subcores=NS),
        scratch_types=[
            pltpu.VMEM_SHARED((NPAD, HHALF), jnp.float32),
            pltpu.VMEM((CHUNK,), jnp.int32),
            pltpu.VMEM((CHUNK,), jnp.int32),
            pltpu.VMEM((CHUNK,), jnp.int32),
            pltpu.VMEM((CHUNK, HHALF), jnp.float32),
            pltpu.VMEM((TAIL,), jnp.int32),
            pltpu.VMEM((TAIL,), jnp.int32),
            pltpu.VMEM((TAIL,), jnp.int32),
            pltpu.VMEM((TAIL, HHALF), jnp.float32),
            pltpu.SemaphoreType.DMA,
            pltpu.SemaphoreType.DMA,
        ],
    )


def _mlp3_body(z_ref, lo_ref, hi_ref, w1_ref, b1_ref,
               w20_ref, w21_ref, w22_ref, b2_ref, h_ref, st_ref):
    i = pl.program_id(0)
    u = z_ref[...] + jnp.concatenate([lo_ref[...], hi_ref[...]], axis=1)
    t = jnp.maximum(
        jnp.dot(u, w1_ref[...], preferred_element_type=jnp.float32)
        + b1_ref[...], 0.0)
    hs = []
    for li, w2_ref in enumerate((w20_ref, w21_ref, w22_ref)):
        ti = t[:, li * H:(li + 1) * H]
        hs.append(jnp.maximum(
            jnp.dot(ti, w2_ref[...], preferred_element_type=jnp.float32)
            + b2_ref[:, li * H:(li + 1) * H], 0.0))
    h = jnp.concatenate(hs, axis=1)
    h_ref[...] = h

    @pl.when(i == 0)
    def _():
        st_ref[...] = jnp.zeros_like(st_ref)

    st_ref[0:1, :] += jnp.sum(h, axis=0, keepdims=True)
    st_ref[1:2, :] += jnp.sum(h * h, axis=0, keepdims=True)


def _mlp3(z, agg_lo, agg_hi, w1c, b1c, w20, w21, w22, b2c):
    return pl.pallas_call(
        _mlp3_body,
        grid=(N // BLK,),
        in_specs=[
            pl.BlockSpec((BLK, D), lambda i: (i, 0)),
            pl.BlockSpec((BLK, HHALF), lambda i: (i, 0)),
            pl.BlockSpec((BLK, HHALF), lambda i: (i, 0)),
            pl.BlockSpec((D, HC), lambda i: (0, 0)),
            pl.BlockSpec((1, HC), lambda i: (0, 0)),
            pl.BlockSpec((H, H), lambda i: (0, 0)),
            pl.BlockSpec((H, H), lambda i: (0, 0)),
            pl.BlockSpec((H, H), lambda i: (0, 0)),
            pl.BlockSpec((1, HC), lambda i: (0, 0)),
        ],
        out_specs=[
            pl.BlockSpec((BLK, HC), lambda i: (i, 0)),
            pl.BlockSpec((8, HC), lambda i: (0, 0)),
        ],
        out_shape=[
            jax.ShapeDtypeStruct((N, HC), jnp.float32),
            jax.ShapeDtypeStruct((8, HC), jnp.float32),
        ],
    )(z, agg_lo, agg_hi, w1c, b1c, w20, w21, w22, b2c)


def _norm_pool_body(h_ref, st_ref, g_ref, b_ref, oh_ref, hbn_ref, pool_ref):
    i = pl.program_id(0)
    st = st_ref[...]
    mean = st[0:1, :] * (1.0 / N)
    var = st[1:2, :] * (1.0 / N) - mean * mean
    rstd = lax.rsqrt(var + 1e-5)
    hbn = (h_ref[...] - mean) * (rstd * g_ref[...]) + b_ref[...]
    hbn_ref[...] = hbn

    @pl.when(i == 0)
    def _():
        pool_ref[...] = jnp.zeros_like(pool_ref)

    pool_ref[...] += lax.dot_general(
        oh_ref[...], hbn, (((0,), (0,)), ((), ())),
        preferred_element_type=jnp.float32)


def _norm_pool(h, st, gamma, beta, onehot):
    return pl.pallas_call(
        _norm_pool_body,
        grid=(N // BLK,),
        in_specs=[
            pl.BlockSpec((BLK, HC), lambda i: (i, 0)),
            pl.BlockSpec((8, HC), lambda i: (0, 0)),
            pl.BlockSpec((1, HC), lambda i: (0, 0)),
            pl.BlockSpec((1, HC), lambda i: (0, 0)),
            pl.BlockSpec((BLK, G), lambda i: (i, 0)),
        ],
        out_specs=[
            pl.BlockSpec((BLK, HC), lambda i: (i, 0)),
            pl.BlockSpec((G, HC), lambda i: (0, 0)),
        ],
        out_shape=[
            jax.ShapeDtypeStruct((N, HC), jnp.float32),
            jax.ShapeDtypeStruct((G, HC), jnp.float32),
        ],
    )(h, st, gamma, beta, onehot)


def kernel(x, edge_index, batch, w1_0, b1_0, w2_0, b2_0, gamma_0, beta_0,
           w1_1, b1_1, w2_1, b2_1, gamma_1, beta_1,
           w1_2, b1_2, w2_2, b2_2, gamma_2, beta_2):
    src = edge_index[0]
    dst = edge_index[1]
    zeros_pad = jnp.zeros((NPAD, HHALF), jnp.float32)
    onehot = (batch[:, None] == jnp.arange(G, dtype=batch.dtype)[None, :]
              ).astype(jnp.float32)

    agg2 = _sc_agg_kernel()(x.reshape(2 * N, HHALF), src, dst, zeros_pad)

    w1c = jnp.concatenate([w1_0, w1_1, w1_2], axis=1)
    b1c = jnp.concatenate([b1_0, b1_1, b1_2]).reshape(1, HC)
    b2c = jnp.concatenate([b2_0, b2_1, b2_2]).reshape(1, HC)
    gmc = jnp.concatenate([gamma_0, gamma_1, gamma_2]).reshape(1, HC)
    btc = jnp.concatenate([beta_0, beta_1, beta_2]).reshape(1, HC)

    h_cat, st = _mlp3(x, agg2[0, :N], agg2[1, :N],
                      w1c, b1c, w2_0, w2_1, w2_2, b2c)
    z_cat, g_cat = _norm_pool(h_cat, st, gmc, btc, onehot)
    return z_cat, g_cat


# speedup vs baseline: 1.6624x; 1.1342x over previous
import functools

import jax
import jax.numpy as jnp
from jax import lax
from jax.experimental import pallas as pl
from jax.experimental.pallas import tpu as pltpu
from jax.experimental.pallas import tpu_sc as plsc

N = 10000
E = 160000
D = 256
H = 256
G = 64
L = 3
HC = H * L
HHALF = 128

NC = 2
NS = 16
NPAD = 10240
ROWS_PER_TILE = NPAD // NS
EDGES_PER_SUB = E // NS
CHUNK = 128
NFULL = EDGES_PER_SUB // CHUNK
TAIL = EDGES_PER_SUB - NFULL * CHUNK

BLK = 1000


def _sc_agg_body(z2, src_h, dst_h, zeros_h, out,
                 acc, src_v, gidx_v, dst_v, rows_v,
                 src_t, gidx_t, dst_t, rows_t, sem, semd, sems):
    c = lax.axis_index("c")
    s = lax.axis_index("s")
    row0 = s * ROWS_PER_TILE
    pltpu.sync_copy(zeros_h.at[pl.ds(row0, ROWS_PER_TILE)],
                    acc.at[pl.ds(row0, ROWS_PER_TILE)])
    plsc.subcore_barrier()

    base0 = s * EDGES_PER_SUB
    pltpu.sync_copy(src_h.at[pl.ds(base0, CHUNK)], src_v)

    def chunk(j, carry):
        base = base0 + j * CHUNK

        @pl.when(j > 0)
        def _():
            pltpu.make_async_copy(src_h.at[pl.ds(base, CHUNK)], src_v,
                                  sems).wait()

        pltpu.async_copy(dst_h.at[pl.ds(base, CHUNK)], dst_v, semd)
        for i in range(CHUNK // 16):
            sl = pl.ds(i * 16, 16)
            gidx_v[sl] = src_v[sl] * 2 + c

        @pl.when(j + 1 < NFULL)
        def _():
            pltpu.async_copy(src_h.at[pl.ds(base + CHUNK, CHUNK)], src_v,
                             sems)

        pltpu.async_copy(z2.at[gidx_v], rows_v, sem).wait()
        pltpu.make_async_copy(dst_h.at[pl.ds(base, CHUNK)], dst_v,
                              semd).wait()
        pltpu.sync_copy(rows_v, acc.at[dst_v], add=True)
        return carry

    lax.fori_loop(0, NFULL, chunk, 0)

    base = base0 + NFULL * CHUNK
    pltpu.sync_copy(src_h.at[pl.ds(base, TAIL)], src_t)
    gidx_t[...] = src_t[...] * 2 + c
    pltpu.async_copy(z2.at[gidx_t], rows_t, sem).wait()
    pltpu.sync_copy(dst_h.at[pl.ds(base, TAIL)], dst_t)
    pltpu.sync_copy(rows_t, acc.at[dst_t], add=True)

    plsc.subcore_barrier()

    pltpu.sync_copy(acc.at[pl.ds(row0, ROWS_PER_TILE)],
                    out.at[c, pl.ds(row0, ROWS_PER_TILE)])


@functools.lru_cache(maxsize=None)
def _sc_agg_kernel():
    return pl.kernel(
        _sc_agg_body,
        out_type=jax.ShapeDtypeStruct((NC, NPAD, HHALF), jnp.float32),
        mesh=plsc.VectorSubcoreMesh(core_axis_name="c", subcore_axis_name="s",
                                    num_cores=NC, num_subcores=NS),
        scratch_types=[
            pltpu.VMEM_SHARED((NPAD, HHALF), jnp.float32),
            pltpu.VMEM((CHUNK,), jnp.int32),
            pltpu.VMEM((CHUNK,), jnp.int32),
            pltpu.VMEM((CHUNK,), jnp.int32),
            pltpu.VMEM((CHUNK, HHALF), jnp.float32),
            pltpu.VMEM((TAIL,), jnp.int32),
            pltpu.VMEM((TAIL,), jnp.int32),
            pltpu.VMEM((TAIL,), jnp.int32),
            pltpu.VMEM((TAIL, HHALF), jnp.float32),
            pltpu.SemaphoreType.DMA,
            pltpu.SemaphoreType.DMA,
            pltpu.SemaphoreType.DMA,
        ],
    )


def _mlp3_body(z_ref, lo_ref, hi_ref, w1_ref, b1_ref,
               w20_ref, w21_ref, w22_ref, b2_ref, h_ref, st_ref):
    i = pl.program_id(0)
    u = z_ref[...] + jnp.concatenate([lo_ref[...], hi_ref[...]], axis=1)
    t = jnp.maximum(
        jnp.dot(u, w1_ref[...], preferred_element_type=jnp.float32)
        + b1_ref[...], 0.0)
    hs = []
    for li, w2_ref in enumerate((w20_ref, w21_ref, w22_ref)):
        ti = t[:, li * H:(li + 1) * H]
        hs.append(jnp.maximum(
            jnp.dot(ti, w2_ref[...], preferred_element_type=jnp.float32)
            + b2_ref[:, li * H:(li + 1) * H], 0.0))
    h = jnp.concatenate(hs, axis=1)
    h_ref[...] = h

    @pl.when(i == 0)
    def _():
        st_ref[...] = jnp.zeros_like(st_ref)

    st_ref[0:1, :] += jnp.sum(h, axis=0, keepdims=True)
    st_ref[1:2, :] += jnp.sum(h * h, axis=0, keepdims=True)


def _mlp3(z, agg_lo, agg_hi, w1c, b1c, w20, w21, w22, b2c):
    return pl.pallas_call(
        _mlp3_body,
        grid=(N // BLK,),
        in_specs=[
            pl.BlockSpec((BLK, D), lambda i: (i, 0)),
            pl.BlockSpec((BLK, HHALF), lambda i: (i, 0)),
            pl.BlockSpec((BLK, HHALF), lambda i: (i, 0)),
            pl.BlockSpec((D, HC), lambda i: (0, 0)),
            pl.BlockSpec((1, HC), lambda i: (0, 0)),
            pl.BlockSpec((H, H), lambda i: (0, 0)),
            pl.BlockSpec((H, H), lambda i: (0, 0)),
            pl.BlockSpec((H, H), lambda i: (0, 0)),
            pl.BlockSpec((1, HC), lambda i: (0, 0)),
        ],
        out_specs=[
            pl.BlockSpec((BLK, HC), lambda i: (i, 0)),
            pl.BlockSpec((8, HC), lambda i: (0, 0)),
        ],
        out_shape=[
            jax.ShapeDtypeStruct((N, HC), jnp.float32),
            jax.ShapeDtypeStruct((8, HC), jnp.float32),
        ],
    )(z, agg_lo, agg_hi, w1c, b1c, w20, w21, w22, b2c)


def _norm_pool_body(h_ref, st_ref, g_ref, b_ref, oh_ref, hbn_ref, pool_ref):
    i = pl.program_id(0)
    st = st_ref[...]
    mean = st[0:1, :] * (1.0 / N)
    var = st[1:2, :] * (1.0 / N) - mean * mean
    rstd = lax.rsqrt(var + 1e-5)
    hbn = (h_ref[...] - mean) * (rstd * g_ref[...]) + b_ref[...]
    hbn_ref[...] = hbn

    @pl.when(i == 0)
    def _():
        pool_ref[...] = jnp.zeros_like(pool_ref)

    pool_ref[...] += lax.dot_general(
        oh_ref[...], hbn, (((0,), (0,)), ((), ())),
        preferred_element_type=jnp.float32)


def _norm_pool(h, st, gamma, beta, onehot):
    return pl.pallas_call(
        _norm_pool_body,
        grid=(N // BLK,),
        in_specs=[
            pl.BlockSpec((BLK, HC), lambda i: (i, 0)),
            pl.BlockSpec((8, HC), lambda i: (0, 0)),
            pl.BlockSpec((1, HC), lambda i: (0, 0)),
            pl.BlockSpec((1, HC), lambda i: (0, 0)),
            pl.BlockSpec((BLK, G), lambda i: (i, 0)),
        ],
        out_specs=[
            pl.BlockSpec((BLK, HC), lambda i: (i, 0)),
            pl.BlockSpec((G, HC), lambda i: (0, 0)),
        ],
        out_shape=[
            jax.ShapeDtypeStruct((N, HC), jnp.float32),
            jax.ShapeDtypeStruct((G, HC), jnp.float32),
        ],
    )(h, st, gamma, beta, onehot)


def kernel(x, edge_index, batch, w1_0, b1_0, w2_0, b2_0, gamma_0, beta_0,
           w1_1, b1_1, w2_1, b2_1, gamma_1, beta_1,
           w1_2, b1_2, w2_2, b2_2, gamma_2, beta_2):
    src = edge_index[0]
    dst = edge_index[1]
    zeros_pad = jnp.zeros((NPAD, HHALF), jnp.float32)
    onehot = (batch[:, None] == jnp.arange(G, dtype=batch.dtype)[None, :]
              ).astype(jnp.float32)

    agg2 = _sc_agg_kernel()(x.reshape(2 * N, HHALF), src, dst, zeros_pad)

    w1c = jnp.concatenate([w1_0, w1_1, w1_2], axis=1)
    b1c = jnp.concatenate([b1_0, b1_1, b1_2]).reshape(1, HC)
    b2c = jnp.concatenate([b2_0, b2_1, b2_2]).reshape(1, HC)
    gmc = jnp.concatenate([gamma_0, gamma_1, gamma_2]).reshape(1, HC)
    btc = jnp.concatenate([beta_0, beta_1, beta_2]).reshape(1, HC)

    h_cat, st = _mlp3(x, agg2[0, :N], agg2[1, :N],
                      w1c, b1c, w2_0, w2_1, w2_2, b2c)
    z_cat, g_cat = _norm_pool(h_cat, st, gmc, btc, onehot)
    return z_cat, g_cat
